# baseline (speedup 1.0000x reference)
import functools

import jax
import jax.numpy as jnp
from jax import lax
from jax.experimental import pallas as pl
from jax.experimental.pallas import tpu as pltpu
from jax.experimental.pallas import tpu_sc as plsc

NC = 2
NS = 16
NW = NC * NS
LANES = 16

B, S, D = 4, 2048, 768
POS_PER_W = S // NW
K = 8
NQ = POS_PER_W // K
COLS = D // LANES
CHALF = COLS // 2
NBUF = 3


def _embed_body(ids_hbm, wte_hbm, wpe_hbm, out_hbm,
                idx_v, bufs, slabs, sem_i,
                sem_p0, sem_p1, sem_p2,
                sem_g0, sem_g1, sem_g2,
                sem_o0, sem_o1, sem_o2):
  cid = lax.axis_index("c")
  sid = lax.axis_index("s")
  wid = sid * NC + cid
  pos0 = wid * POS_PER_W

  sem_p = (sem_p0, sem_p1, sem_p2)
  sem_g = (sem_g0, sem_g1, sem_g2)
  sem_o = (sem_o0, sem_o1, sem_o2)

  idx_cps = [
      pltpu.async_copy(
          ids_hbm.at[b, pl.ds(pos0, POS_PER_W)], idx_v.at[b], sem_i)
      for b in range(B)
  ]

  def issue_group(q):
    gp = q % NBUF
    slab_cp = pltpu.async_copy(
        wpe_hbm.at[pl.ds(pos0 + q * K, K)], slabs.at[gp], sem_p[gp])
    g_cps = [
        pltpu.async_copy(
            wte_hbm.at[idx_v.at[b, pl.ds(q * K, K)]], bufs.at[gp, b],
            sem_g[gp])
        for b in range(B)
    ]
    return (slab_cp, g_cps)

  for cp in idx_cps:
    cp.wait()
  pend = {0: issue_group(0), 1: issue_group(1)}
  wbs = {}
  for q in range(NQ):
    gp = q % NBUF
    slab_cp, g_cps = pend[q]
    slab_cp.wait()
    for cp in g_cps:
      cp.wait()

    def row_body(r, _, gp=gp):
      for half in range(2):
        base = half * CHALF * LANES
        wrow = [slabs[gp, r, pl.ds(base + j * LANES, LANES)]
                for j in range(CHALF)]
        for b in range(B):
          for j in range(CHALF):
            sl = pl.ds(base + j * LANES, LANES)
            bufs[gp, b, r, sl] = bufs[gp, b, r, sl] + wrow[j]
      return 0

    lax.fori_loop(0, K, row_body, 0)

    wbs[q] = [
        pltpu.async_copy(
            bufs.at[gp, b], out_hbm.at[pl.ds(b * S + pos0 + q * K, K)],
            sem_o[gp])
        for b in range(B)
    ]
    if q + 2 < NQ:
      if q >= 1:
        for cp in wbs[q - 1]:
          cp.wait()
      pend[q + 2] = issue_group(q + 2)

  for q in range(NQ - NBUF, NQ):
    for cp in wbs[q]:
      cp.wait()


@jax.jit
def _embed(ids, wte, wpe):
  mesh = plsc.VectorSubcoreMesh(core_axis_name="c", subcore_axis_name="s")
  f = pl.kernel(
      _embed_body,
      out_type=jax.ShapeDtypeStruct((B * S, D), jnp.float32),
      mesh=mesh,
      scratch_types=[
          pltpu.VMEM((B, POS_PER_W), jnp.int32),
          pltpu.VMEM((NBUF, B, K, D), jnp.float32),
          pltpu.VMEM((NBUF, K, D), jnp.float32),
          pltpu.SemaphoreType.DMA,
          pltpu.SemaphoreType.DMA,
          pltpu.SemaphoreType.DMA,
          pltpu.SemaphoreType.DMA,
          pltpu.SemaphoreType.DMA,
          pltpu.SemaphoreType.DMA,
          pltpu.SemaphoreType.DMA,
          pltpu.SemaphoreType.DMA,
          pltpu.SemaphoreType.DMA,
          pltpu.SemaphoreType.DMA,
      ],
  )
  return f(ids, wte, wpe)


def kernel(input_ids, wte, wpe):
  out = _embed(input_ids.astype(jnp.int32), wte, wpe)
  return out.reshape(B, S, D)


# speedup vs baseline: 1.0364x; 1.0364x over previous
import functools

import jax
import jax.numpy as jnp
from jax import lax
from jax.experimental import pallas as pl
from jax.experimental.pallas import tpu as pltpu
from jax.experimental.pallas import tpu_sc as plsc

NC = 2
NS = 16
NW = NC * NS
LANES = 16

B, S, D = 4, 2048, 768
POS_PER_W = S // NW
K = 16
NQ = POS_PER_W // K
GROWS = B * K
COLS = D // LANES
CHALF = COLS // 2


def _embed_body(ids_hbm, wte_hbm, wpe_hbm, out_hbm,
                idx_v, bufs, slabs, sem_i, sem_p,
                sem_g0, sem_g1, sem_o0, sem_o1):
  cid = lax.axis_index("c")
  sid = lax.axis_index("s")
  wid = sid * NC + cid
  pos0 = wid * POS_PER_W

  sem_g = (sem_g0, sem_g1)
  sem_o = (sem_o0, sem_o1)

  idx_cps = []
  for q in range(NQ):
    for b in range(B):
      idx_cps.append(pltpu.async_copy(
          ids_hbm.at[b, pl.ds(pos0 + q * K, K)],
          idx_v.at[q, pl.ds(b * K, K)], sem_i))

  def issue_group(q):
    gp = q & 1
    slab_cp = pltpu.async_copy(
        wpe_hbm.at[pl.ds(pos0 + q * K, K)], slabs.at[gp], sem_p)
    g_cp = pltpu.async_copy(
        wte_hbm.at[idx_v.at[q]], bufs.at[gp], sem_g[gp])
    return (slab_cp, g_cp)

  for cp in idx_cps:
    cp.wait()

  pend = {0: issue_group(0), 1: issue_group(1)}
  wbs = {}
  for q in range(NQ):
    gp = q & 1
    slab_cp, g_cp = pend[q]
    slab_cp.wait()
    g_cp.wait()

    def row_body(r, _, gp=gp):
      for half in range(2):
        base = half * CHALF * LANES
        wrow = [slabs[gp, r, pl.ds(base + j * LANES, LANES)]
                for j in range(CHALF)]
        for b in range(B):
          row = b * K + r
          for j in range(CHALF):
            sl = pl.ds(base + j * LANES, LANES)
            bufs[gp, row, sl] = bufs[gp, row, sl] + wrow[j]
      return 0

    lax.fori_loop(0, K, row_body, 0)

    wbs[q] = [
        pltpu.async_copy(
            bufs.at[gp, pl.ds(b * K, K)],
            out_hbm.at[pl.ds(b * S + pos0 + q * K, K)], sem_o[gp])
        for b in range(B)
    ]
    if q + 2 < NQ:
      for cp in wbs[q]:
        cp.wait()
      pend[q + 2] = issue_group(q + 2)

  for q in (NQ - 2, NQ - 1):
    for cp in wbs[q]:
      cp.wait()


@jax.jit
def _embed(ids, wte, wpe):
  mesh = plsc.VectorSubcoreMesh(core_axis_name="c", subcore_axis_name="s")
  f = pl.kernel(
      _embed_body,
      out_type=jax.ShapeDtypeStruct((B * S, D), jnp.float32),
      mesh=mesh,
      scratch_types=[
          pltpu.VMEM((NQ, GROWS), jnp.int32),
          pltpu.VMEM((2, GROWS, D), jnp.float32),
          pltpu.VMEM((2, K, D), jnp.float32),
          pltpu.SemaphoreType.DMA,
          pltpu.SemaphoreType.DMA,
          pltpu.SemaphoreType.DMA,
          pltpu.SemaphoreType.DMA,
          pltpu.SemaphoreType.DMA,
          pltpu.SemaphoreType.DMA,
      ],
  )
  return f(ids, wte, wpe)


def kernel(input_ids, wte, wpe):
  out = _embed(input_ids.astype(jnp.int32), wte, wpe)
  return out.reshape(B, S, D)


# speedup vs baseline: 1.0434x; 1.0068x over previous
import functools

import jax
import jax.numpy as jnp
from jax import lax
from jax.experimental import pallas as pl
from jax.experimental.pallas import tpu as pltpu
from jax.experimental.pallas import tpu_sc as plsc

NC = 2
NS = 16
NW = NC * NS
LANES = 16

B, S, D = 4, 2048, 768
POS_PER_W = S // NW
K = 16
NQ = POS_PER_W // K
NCH = 2 * NQ
CROWS = 2 * K
COLS = D // LANES
CHALF = COLS // 2
NBUF = 4


def _embed_body(ids_hbm, wte_hbm, wpe_hbm, out_hbm,
                idx_v, bufs, slabs, sem_i, sem_p,
                sem_g0, sem_g1, sem_g2, sem_g3,
                sem_o0, sem_o1, sem_o2, sem_o3):
  cid = lax.axis_index("c")
  sid = lax.axis_index("s")
  wid = sid * NC + cid
  pos0 = wid * POS_PER_W

  sem_g = (sem_g0, sem_g1, sem_g2, sem_g3)
  sem_o = (sem_o0, sem_o1, sem_o2, sem_o3)

  idx_cps = []
  for n in range(NCH):
    q, hb = n // 2, n % 2
    for b2 in range(2):
      idx_cps.append(pltpu.async_copy(
          ids_hbm.at[hb * 2 + b2, pl.ds(pos0 + q * K, K)],
          idx_v.at[n, pl.ds(b2 * K, K)], sem_i))

  slab_cps = {}

  def issue_chunk(n):
    gp = n % NBUF
    q, hb = n // 2, n % 2
    if hb == 1:
      slab_cps[q] = pltpu.async_copy(
          wpe_hbm.at[pl.ds(pos0 + q * K, K)], slabs.at[q & 1], sem_p)
    return pltpu.async_copy(
        wte_hbm.at[idx_v.at[n]], bufs.at[gp], sem_g[gp])

  for cp in idx_cps:
    cp.wait()

  pend = {n: issue_chunk(n) for n in range(NBUF)}
  wbs = {}
  for n in range(NCH):
    gp = n % NBUF
    q, hb = n // 2, n % 2
    if hb == 0:
      slab_cps[q].wait()
    pend[n].wait()

    def row_body(r, _, gp=gp, sp=q & 1):
      for half in range(2):
        base = half * CHALF * LANES
        wrow = [slabs[sp, r, pl.ds(base + j * LANES, LANES)]
                for j in range(CHALF)]
        for b2 in range(2):
          row = b2 * K + r
          for j in range(CHALF):
            sl = pl.ds(base + j * LANES, LANES)
            bufs[gp, row, sl] = bufs[gp, row, sl] + wrow[j]
      return 0

    lax.fori_loop(0, K, row_body, 0)

    wbs[n] = [
        pltpu.async_copy(
            bufs.at[gp, pl.ds(b2 * K, K)],
            out_hbm.at[pl.ds((hb * 2 + b2) * S + pos0 + q * K, K)],
            sem_o[gp])
        for b2 in range(2)
    ]
    if n + NBUF < NCH:
      for cp in wbs[n]:
        cp.wait()
      pend[n + NBUF] = issue_chunk(n + NBUF)

  for n in range(NCH - NBUF, NCH):
    for cp in wbs[n]:
      cp.wait()


@jax.jit
def _embed(ids, wte, wpe):
  mesh = plsc.VectorSubcoreMesh(core_axis_name="c", subcore_axis_name="s")
  f = pl.kernel(
      _embed_body,
      out_type=jax.ShapeDtypeStruct((B * S, D), jnp.float32),
      mesh=mesh,
      scratch_types=[
          pltpu.VMEM((NCH, CROWS), jnp.int32),
          pltpu.VMEM((NBUF, CROWS, D), jnp.float32),
          pltpu.VMEM((2, K, D), jnp.float32),
          pltpu.SemaphoreType.DMA,
          pltpu.SemaphoreType.DMA,
          pltpu.SemaphoreType.DMA,
          pltpu.SemaphoreType.DMA,
          pltpu.SemaphoreType.DMA,
          pltpu.SemaphoreType.DMA,
          pltpu.SemaphoreType.DMA,
          pltpu.SemaphoreType.DMA,
          pltpu.SemaphoreType.DMA,
          pltpu.SemaphoreType.DMA,
      ],
  )
  return f(ids, wte, wpe)


def kernel(input_ids, wte, wpe):
  out = _embed(input_ids.astype(jnp.int32), wte, wpe)
  return out.reshape(B, S, D)


# speedup vs baseline: 1.2001x; 1.1502x over previous
import functools

import jax
import jax.numpy as jnp
from jax import lax
from jax.experimental import pallas as pl
from jax.experimental.pallas import tpu as pltpu
from jax.experimental.pallas import tpu_sc as plsc

NC = 2
NS = 16
NW = NC * NS
LANES = 16

B, S, D = 4, 2048, 768
POS_PER_W = S // NW
K = 16
NCH = B * POS_PER_W // (2 * K)
CROWS = 2 * K
COLS = D // LANES
CHALF = COLS // 2
NBUF = 3


def _embed_body(ids_hbm, wte_hbm, wpe_hbm, out_hbm,
                idx_v, bufs, wpe_v, sem_i, sem_p, sem_g, sem_o):
  cid = lax.axis_index("c")
  sid = lax.axis_index("s")
  wid = sid * NC + cid
  pos0 = wid * POS_PER_W

  cp_wpe = pltpu.async_copy(wpe_hbm.at[pl.ds(pos0, POS_PER_W)], wpe_v, sem_p)
  idx_cps = []
  for n in range(NCH):
    q, hb = n // 2, n % 2
    for b2 in range(2):
      idx_cps.append(pltpu.async_copy(
          ids_hbm.at[hb * 2 + b2, pl.ds(pos0 + q * K, K)],
          idx_v.at[n, pl.ds(b2 * K, K)], sem_i))
  for cp in idx_cps:
    cp.wait()

  def issue_gather(n):
    return pltpu.async_copy(
        wte_hbm.at[idx_v.at[n]], bufs.at[n % NBUF], sem_g)

  issue_gather(0)
  issue_gather(1)
  cp_wpe.wait()

  def wb_pair(n):
    q, hb = n // 2, n % 2
    return [(bufs.at[n % NBUF, pl.ds(b2 * K, K)],
             out_hbm.at[pl.ds((hb * 2 + b2) * S + pos0 + q * K, K)])
            for b2 in range(2)]

  def chunk_body(n, _):
    gp = n % NBUF
    q = n // 2
    hb = n % 2
    pltpu.make_async_copy(
        wte_hbm.at[idx_v.at[n]], bufs.at[gp], sem_g).wait()

    def row_body(r, _):
      wr = q * K + r
      for half in range(2):
        base = half * CHALF * LANES
        wrow = [wpe_v[wr, pl.ds(base + j * LANES, LANES)]
                for j in range(CHALF)]
        for b2 in range(2):
          row = b2 * K + r
          for j in range(CHALF):
            sl = pl.ds(base + j * LANES, LANES)
            bufs[gp, row, sl] = bufs[gp, row, sl] + wrow[j]
      return 0

    lax.fori_loop(0, K, row_body, 0)

    for b2 in range(2):
      pltpu.async_copy(
          bufs.at[gp, pl.ds(b2 * K, K)],
          out_hbm.at[pl.ds((hb * 2 + b2) * S + pos0 + q * K, K)], sem_o)

    @pl.when(n >= 1)
    def _():
      for b2 in range(2):
        pltpu.make_async_copy(
            bufs.at[(n - 1) % NBUF, pl.ds(b2 * K, K)],
            out_hbm.at[pl.ds(pos0, K)], sem_o).wait()

    @pl.when(n + 2 < NCH)
    def _():
      pltpu.async_copy(
          wte_hbm.at[idx_v.at[n + 2]], bufs.at[(n + 2) % NBUF], sem_g)

    return 0

  lax.fori_loop(0, NCH, chunk_body, 0)

  for b2 in range(2):
    pltpu.make_async_copy(
        bufs.at[(NCH - 1) % NBUF, pl.ds(b2 * K, K)],
        out_hbm.at[pl.ds(pos0, K)], sem_o).wait()


@jax.jit
def _embed(ids, wte, wpe):
  mesh = plsc.VectorSubcoreMesh(core_axis_name="c", subcore_axis_name="s")
  f = pl.kernel(
      _embed_body,
      out_type=jax.ShapeDtypeStruct((B * S, D), jnp.float32),
      mesh=mesh,
      scratch_types=[
          pltpu.VMEM((NCH, CROWS), jnp.int32),
          pltpu.VMEM((NBUF, CROWS, D), jnp.float32),
          pltpu.VMEM((POS_PER_W, D), jnp.float32),
          pltpu.SemaphoreType.DMA,
          pltpu.SemaphoreType.DMA,
          pltpu.SemaphoreType.DMA,
          pltpu.SemaphoreType.DMA,
      ],
  )
  return f(ids, wte, wpe)


def kernel(input_ids, wte, wpe):
  out = _embed(input_ids.astype(jnp.int32), wte, wpe)
  return out.reshape(B, S, D)


# speedup vs baseline: 1.2010x; 1.0008x over previous
import functools

import jax
import jax.numpy as jnp
from jax import lax
from jax.experimental import pallas as pl
from jax.experimental.pallas import tpu as pltpu
from jax.experimental.pallas import tpu_sc as plsc

NC = 2
NS = 16
NW = NC * NS
LANES = 16

B, S, D = 4, 2048, 768
POS_PER_W = S // NW
K = 16
NCH = B * POS_PER_W // (2 * K)
CROWS = 2 * K
COLS = D // LANES
CHALF = COLS // 2
NBUF = 3


def _embed_body(ids_hbm, wte_hbm, wpe_hbm, out_hbm,
                idx_v, bufs, wpe_v, sem_i, sem_p, sem_g, sem_o):
  cid = lax.axis_index("c")
  sid = lax.axis_index("s")
  wid = sid * NC + cid
  pos0 = wid * POS_PER_W

  cp_wpe = pltpu.async_copy(wpe_hbm.at[pl.ds(pos0, POS_PER_W)], wpe_v, sem_p)
  idx_cps = []
  for n in range(NCH):
    q, hb = n // 2, n % 2
    for b2 in range(2):
      idx_cps.append(pltpu.async_copy(
          ids_hbm.at[hb * 2 + b2, pl.ds(pos0 + q * K, K)],
          idx_v.at[n, pl.ds(b2 * K, K)], sem_i))

  def issue_gather(n):
    return pltpu.async_copy(
        wte_hbm.at[idx_v.at[n]], bufs.at[n % NBUF], sem_g)

  idx_cps[0].wait()
  idx_cps[1].wait()
  issue_gather(0)
  idx_cps[2].wait()
  idx_cps[3].wait()
  issue_gather(1)
  for cp in idx_cps[4:]:
    cp.wait()
  cp_wpe.wait()

  def wb_pair(n):
    q, hb = n // 2, n % 2
    return [(bufs.at[n % NBUF, pl.ds(b2 * K, K)],
             out_hbm.at[pl.ds((hb * 2 + b2) * S + pos0 + q * K, K)])
            for b2 in range(2)]

  def chunk_body(n, _):
    gp = n % NBUF
    q = n // 2
    hb = n % 2
    pltpu.make_async_copy(
        wte_hbm.at[idx_v.at[n]], bufs.at[gp], sem_g).wait()

    def row_body(r, _):
      wr = q * K + r

      def half_body(half, _):
        base = half * (CHALF * LANES)
        wrow = [wpe_v[wr, pl.ds(base + j * LANES, LANES)]
                for j in range(CHALF)]
        for b2 in range(2):
          row = b2 * K + r
          for j in range(CHALF):
            sl = pl.ds(base + j * LANES, LANES)
            bufs[gp, row, sl] = bufs[gp, row, sl] + wrow[j]
        return 0

      return lax.fori_loop(0, 2, half_body, 0)

    lax.fori_loop(0, K, row_body, 0)

    for b2 in range(2):
      pltpu.async_copy(
          bufs.at[gp, pl.ds(b2 * K, K)],
          out_hbm.at[pl.ds((hb * 2 + b2) * S + pos0 + q * K, K)], sem_o)

    @pl.when(n >= 1)
    def _():
      for b2 in range(2):
        pltpu.make_async_copy(
            bufs.at[(n - 1) % NBUF, pl.ds(b2 * K, K)],
            out_hbm.at[pl.ds(pos0, K)], sem_o).wait()

    @pl.when(n + 2 < NCH)
    def _():
      pltpu.async_copy(
          wte_hbm.at[idx_v.at[n + 2]], bufs.at[(n + 2) % NBUF], sem_g)

    return 0

  lax.fori_loop(0, NCH, chunk_body, 0)

  for b2 in range(2):
    pltpu.make_async_copy(
        bufs.at[(NCH - 1) % NBUF, pl.ds(b2 * K, K)],
        out_hbm.at[pl.ds(pos0, K)], sem_o).wait()


@jax.jit
def _embed(ids, wte, wpe):
  mesh = plsc.VectorSubcoreMesh(core_axis_name="c", subcore_axis_name="s")
  f = pl.kernel(
      _embed_body,
      out_type=jax.ShapeDtypeStruct((B * S, D), jnp.float32),
      mesh=mesh,
      scratch_types=[
          pltpu.VMEM((NCH, CROWS), jnp.int32),
          pltpu.VMEM((NBUF, CROWS, D), jnp.float32),
          pltpu.VMEM((POS_PER_W, D), jnp.float32),
          pltpu.SemaphoreType.DMA,
          pltpu.SemaphoreType.DMA,
          pltpu.SemaphoreType.DMA,
          pltpu.SemaphoreType.DMA,
      ],
  )
  return f(ids, wte, wpe)


def kernel(input_ids, wte, wpe):
  out = _embed(input_ids.astype(jnp.int32), wte, wpe)
  return out.reshape(B, S, D)


# speedup vs baseline: 1.2015x; 1.0004x over previous
import functools

import jax
import jax.numpy as jnp
from jax import lax
from jax.experimental import pallas as pl
from jax.experimental.pallas import tpu as pltpu
from jax.experimental.pallas import tpu_sc as plsc

NC = 2
NS = 16
NW = NC * NS
LANES = 16

B, S, D = 4, 2048, 768
POS_PER_W = S // NW
K = 16
NCH = B * POS_PER_W // (2 * K)
CROWS = 2 * K
COLS = D // LANES
CHALF = COLS // 2
NBUF = 3


def _embed_body(ids_hbm, wte_hbm, wpe_hbm, out_hbm,
                idx_v, bufs, wpe_v, sem_i, sem_p, sem_g, sem_o):
  cid = lax.axis_index("c")
  sid = lax.axis_index("s")
  wid = sid * NC + cid
  pos0 = wid * POS_PER_W

  cp_wpe = pltpu.async_copy(wpe_hbm.at[pl.ds(pos0, POS_PER_W)], wpe_v, sem_p)
  idx_cps = []
  for n in range(NCH):
    q, hb = n // 2, n % 2
    for b2 in range(2):
      idx_cps.append(pltpu.async_copy(
          ids_hbm.at[hb * 2 + b2, pl.ds(pos0 + q * K, K)],
          idx_v.at[n, pl.ds(b2 * K, K)], sem_i))

  def issue_gather(n):
    return pltpu.async_copy(
        wte_hbm.at[idx_v.at[n]], bufs.at[n % NBUF], sem_g)

  idx_cps[0].wait()
  idx_cps[1].wait()
  issue_gather(0)
  idx_cps[2].wait()
  idx_cps[3].wait()
  issue_gather(1)
  for cp in idx_cps[4:]:
    cp.wait()
  cp_wpe.wait()

  def wb_pair(n):
    q, hb = n // 2, n % 2
    return [(bufs.at[n % NBUF, pl.ds(b2 * K, K)],
             out_hbm.at[pl.ds((hb * 2 + b2) * S + pos0 + q * K, K)])
            for b2 in range(2)]

  def chunk_body(n, _):
    gp = n % NBUF
    q = n // 2
    hb = n % 2
    pltpu.make_async_copy(
        wte_hbm.at[idx_v.at[n]], bufs.at[gp], sem_g).wait()

    def row_body(r, _):
      wr = q * K + r
      for half in range(2):
        base = half * CHALF * LANES
        wrow = [wpe_v[wr, pl.ds(base + j * LANES, LANES)]
                for j in range(CHALF)]
        for b2 in range(2):
          row = b2 * K + r
          for j in range(CHALF):
            sl = pl.ds(base + j * LANES, LANES)
            bufs[gp, row, sl] = bufs[gp, row, sl] + wrow[j]
      return 0

    lax.fori_loop(0, K, row_body, 0)

    for b2 in range(2):
      pltpu.async_copy(
          bufs.at[gp, pl.ds(b2 * K, K)],
          out_hbm.at[pl.ds((hb * 2 + b2) * S + pos0 + q * K, K)], sem_o)

    @pl.when(n >= 1)
    def _():
      for b2 in range(2):
        pltpu.make_async_copy(
            bufs.at[(n - 1) % NBUF, pl.ds(b2 * K, K)],
            out_hbm.at[pl.ds(pos0, K)], sem_o).wait()

    @pl.when(n + 2 < NCH)
    def _():
      pltpu.async_copy(
          wte_hbm.at[idx_v.at[n + 2]], bufs.at[(n + 2) % NBUF], sem_g)

    return 0

  lax.fori_loop(0, NCH, chunk_body, 0)

  for b2 in range(2):
    pltpu.make_async_copy(
        bufs.at[(NCH - 1) % NBUF, pl.ds(b2 * K, K)],
        out_hbm.at[pl.ds(pos0, K)], sem_o).wait()


@jax.jit
def _embed(ids, wte, wpe):
  mesh = plsc.VectorSubcoreMesh(core_axis_name="c", subcore_axis_name="s")
  f = pl.kernel(
      _embed_body,
      out_type=jax.ShapeDtypeStruct((B * S, D), jnp.float32),
      mesh=mesh,
      scratch_types=[
          pltpu.VMEM((NCH, CROWS), jnp.int32),
          pltpu.VMEM((NBUF, CROWS, D), jnp.float32),
          pltpu.VMEM((POS_PER_W, D), jnp.float32),
          pltpu.SemaphoreType.DMA,
          pltpu.SemaphoreType.DMA,
          pltpu.SemaphoreType.DMA,
          pltpu.SemaphoreType.DMA,
      ],
  )
  return f(ids, wte, wpe)


def kernel(input_ids, wte, wpe):
  out = _embed(input_ids.astype(jnp.int32), wte, wpe)
  return out.reshape(B, S, D)


# speedup vs baseline: 1.2325x; 1.0258x over previous
import functools

import jax
import jax.numpy as jnp
from jax import lax
from jax.experimental import pallas as pl
from jax.experimental.pallas import tpu as pltpu
from jax.experimental.pallas import tpu_sc as plsc

NC = 2
NS = 16
NW = NC * NS
LANES = 16

B, S, D = 4, 2048, 768
POS_PER_W = S // NW
K = 16
NQ = POS_PER_W // K
NCH = 2 * NQ
CROWS = 2 * K
COLS = D // LANES
CHALF = COLS // 2
NBUF = 4


def _embed_body(ids_hbm, wte_hbm, wpe_hbm, out_hbm,
                idx_v, bufs, wpe_s, sem_i, sem_p, sem_g, sem_o):
  cid = lax.axis_index("c")
  sid = lax.axis_index("s")
  wid = sid * NC + cid
  pos0 = wid * POS_PER_W

  idx_cps = []
  for n in range(NCH):
    q, hb = n // 2, n % 2
    for b2 in range(2):
      idx_cps.append(pltpu.async_copy(
          ids_hbm.at[hb * 2 + b2, pl.ds(pos0 + q * K, K)],
          idx_v.at[n, pl.ds(b2 * K, K)], sem_i))

  def issue_slab(q):
    return pltpu.async_copy(
        wpe_hbm.at[pl.ds(pos0 + q * K, K)], wpe_s.at[q % 2], sem_p)

  def issue_gather(n):
    return pltpu.async_copy(
        wte_hbm.at[idx_v.at[n]], bufs.at[n % NBUF], sem_g)

  issue_slab(0)
  issue_slab(1)
  for n in range(3):
    idx_cps[2 * n].wait()
    idx_cps[2 * n + 1].wait()
    issue_gather(n)
  for cp in idx_cps[6:]:
    cp.wait()

  def chunk_body(n, _):
    gp = n % NBUF
    q = n // 2
    hb = n % 2
    sp = q % 2

    @pl.when(hb == 0)
    def _():
      pltpu.make_async_copy(
          wpe_hbm.at[pl.ds(pos0, K)], wpe_s.at[0], sem_p).wait()

    pltpu.make_async_copy(
        wte_hbm.at[idx_v.at[n]], bufs.at[gp], sem_g).wait()

    def row_body(r, _):
      for half in range(2):
        base = half * CHALF * LANES
        wrow = [wpe_s[sp, r, pl.ds(base + j * LANES, LANES)]
                for j in range(CHALF)]
        for b2 in range(2):
          row = b2 * K + r
          for j in range(CHALF):
            sl = pl.ds(base + j * LANES, LANES)
            bufs[gp, row, sl] = bufs[gp, row, sl] + wrow[j]
      return 0

    lax.fori_loop(0, K, row_body, 0)

    for b2 in range(2):
      pltpu.async_copy(
          bufs.at[gp, pl.ds(b2 * K, K)],
          out_hbm.at[pl.ds((hb * 2 + b2) * S + pos0 + q * K, K)], sem_o)

    @pl.when(jnp.logical_and(hb == 1, q + 2 < NQ))
    def _():
      pltpu.async_copy(
          wpe_hbm.at[pl.ds(pos0 + (q + 2) * K, K)], wpe_s.at[sp], sem_p)

    @pl.when(n >= 1)
    def _():
      for b2 in range(2):
        pltpu.make_async_copy(
            bufs.at[(n - 1) % NBUF, pl.ds(b2 * K, K)],
            out_hbm.at[pl.ds(pos0, K)], sem_o).wait()

    @pl.when(n + 3 < NCH)
    def _():
      pltpu.async_copy(
          wte_hbm.at[idx_v.at[n + 3]], bufs.at[(n + 3) % NBUF], sem_g)

    return 0

  lax.fori_loop(0, NCH, chunk_body, 0)

  for b2 in range(2):
    pltpu.make_async_copy(
        bufs.at[(NCH - 1) % NBUF, pl.ds(b2 * K, K)],
        out_hbm.at[pl.ds(pos0, K)], sem_o).wait()


@jax.jit
def _embed(ids, wte, wpe):
  mesh = plsc.VectorSubcoreMesh(core_axis_name="c", subcore_axis_name="s")
  f = pl.kernel(
      _embed_body,
      out_type=jax.ShapeDtypeStruct((B * S, D), jnp.float32),
      mesh=mesh,
      scratch_types=[
          pltpu.VMEM((NCH, CROWS), jnp.int32),
          pltpu.VMEM((NBUF, CROWS, D), jnp.float32),
          pltpu.VMEM((2, K, D), jnp.float32),
          pltpu.SemaphoreType.DMA,
          pltpu.SemaphoreType.DMA,
          pltpu.SemaphoreType.DMA,
          pltpu.SemaphoreType.DMA,
      ],
  )
  return f(ids, wte, wpe)


def kernel(input_ids, wte, wpe):
  out = _embed(input_ids.astype(jnp.int32), wte, wpe)
  return out.reshape(B, S, D)
